# Initial kernel scaffold; baseline (speedup 1.0000x reference)
import jax
import jax.numpy as jnp
from jax.experimental import pallas as pl


def kernel(x, pos, edge_index, norm, batch):
    raise NotImplementedError("write your pallas kernel here")


# speedup vs baseline: 1.0256x; 1.0256x over previous
import jax
import jax.numpy as jnp
from jax.experimental import pallas as pl


def _angle(v1, v2):
    c = jnp.cross(v1, v2)
    sq = jnp.sum(c * c, axis=1)
    cross_norm = jnp.sqrt(jnp.where(sq == 0.0, 1.0, sq))
    cross_norm = jnp.where(sq == 0.0, 0.0, cross_norm)
    dot = jnp.sum(v1 * v2, axis=1)
    both_zero = (sq == 0.0) & (dot == 0.0)
    safe_dot = jnp.where(both_zero, 1.0, dot)
    ang = jnp.arctan2(cross_norm, safe_dot)
    return jnp.where(both_zero, 0.0, ang)


def _scale_body(o_ref, s_ref, out_ref):
    blk = o_ref[...]
    scale = s_ref[0, 0]
    col = jax.lax.broadcasted_iota(jnp.int32, blk.shape, 1)
    out_ref[...] = jnp.where(col == 128, blk * scale, blk)


def kernel(x, pos, edge_index, norm, batch):
    N = pos.shape[0]
    row0 = edge_index[0].astype(jnp.int32)
    col0 = edge_index[1].astype(jnp.int32)
    keep = row0 != col0
    n1 = norm[row0]
    n2 = norm[col0]
    d = pos[col0] - pos[row0]
    dist = jnp.sum(d * d, axis=1)
    num_edges = (jnp.sum(keep) + N).astype(dist.dtype)
    total = jnp.sum(dist)
    feats = jnp.concatenate(
        [x[col0],
         jnp.stack([dist, _angle(n1, d), _angle(n2, d), _angle(n1, n2)], axis=1)],
        axis=1)
    init = jnp.concatenate([x, jnp.zeros((N, 4), x.dtype)], axis=1)
    segmax = jax.ops.segment_max(feats, row0, num_segments=N)
    out = jnp.maximum(init, segmax)
    inv_scale = (num_edges / total).astype(jnp.float32).reshape(1, 1)
    out = pl.pallas_call(
        _scale_body,
        out_shape=jax.ShapeDtypeStruct((N, 132), jnp.float32),
        in_specs=[pl.BlockSpec((N, 132), lambda: (0, 0)),
                  pl.BlockSpec((1, 1), lambda: (0, 0))],
        out_specs=pl.BlockSpec((N, 132), lambda: (0, 0)),
    )(out, inv_scale)
    return out


# speedup vs baseline: 3.6585x; 3.5673x over previous
import functools

import jax
import jax.numpy as jnp
from jax import lax
from jax.experimental import pallas as pl
from jax.experimental.pallas import tpu as pltpu
from jax.experimental.pallas import tpu_sc as plsc

N = 10000
E = 320000
D = 128
NC = 2
NS = 16
NP = 10016
RPW = 626
EH = E // NC
C = 1280
B = 32
VEC = 16


def _iota():
    return lax.broadcasted_iota(jnp.int32, (VEC,), 0)


def _surrogate(ax, ay, az, bx, by, bz):
    cx = ay * bz - az * by
    cy = az * bx - ax * bz
    cz = ax * by - ay * bx
    sq = cx * cx + cy * cy + cz * cz
    dot = ax * bx + ay * by + az * bz
    den = dot * dot + sq
    u = -(dot * jnp.abs(dot)) / den
    return jnp.where(den == 0.0, jnp.float32(-1.0), u)


def _sc_body(rowh, colh, xh, pnh,
             outx, outp, sums, counts,
             accx, accp, rowb, colb, mrow, mcol, xb, pnr, pnc, fbuf, svec,
             cvec, sem):
    c = lax.axis_index("c")
    s = lax.axis_index("s")
    w = c * NS + s
    lo = s * RPW
    coff = c * EH
    iota = _iota()
    zero16 = jnp.zeros((VEC,), jnp.float32)

    pltpu.sync_copy(xh.at[pl.ds(lo, RPW)], accx)
    pinit = jnp.where((iota >= 1) & (iota <= 3), jnp.float32(-1.0),
                      jnp.float32(0.0))

    def init_p(n, _):
        accp[pl.ds(n * VEC, VEC)] = pinit
        return 0

    lax.fori_loop(jnp.int32(0), jnp.int32(RPW), init_p, 0)
    for z in range(4, 16):
        fbuf[pl.ds(z * VEC, VEC)] = zero16
    for z in range((C + VEC) // VEC):
        mrow[pl.ds(z * VEC, VEC)] = jnp.zeros((VEC,), jnp.int32)
        mcol[pl.ds(z * VEC, VEC)] = jnp.zeros((VEC,), jnp.int32)

    cj = [jnp.int32(j * VEC) + iota for j in range(D // VEC)]

    def chunk_body(ch, carry):
        kcv, dsv = carry
        pltpu.sync_copy(rowh.at[pl.ds(coff + ch * C, C)], rowb)
        pltpu.sync_copy(colh.at[pl.ds(coff + ch * C, C)], colb)

        def scan_body(i, sc):
            cnt, kcv = sc
            rv = rowb[pl.ds(i * VEC, VEC)]
            cv = colb[pl.ds(i * VEC, VEC)]
            msk = (rv >= lo) & (rv < lo + RPW)
            kcv = kcv + jnp.where(msk & (rv != cv), jnp.int32(1), jnp.int32(0))
            inc = jnp.where(msk, jnp.int32(1), jnp.int32(0))
            for k in (1, 2, 4, 8):
                sh = inc.at[jnp.maximum(iota - k, 0)].get(
                    mode="promise_in_bounds")
                inc = inc + jnp.where(iota >= k, sh, jnp.int32(0))
            pos = jnp.where(msk, cnt + inc - 1, jnp.int32(C) + iota)
            plsc.store_scatter(mrow, [pos], rv)
            plsc.store_scatter(mcol, [pos], cv)
            return cnt + inc[15], kcv

        m, kcv = lax.fori_loop(jnp.int32(0), jnp.int32(C // VEC), scan_body,
                               (jnp.int32(0), kcv))

        def batch_body(b, dsv):
            cpx = pltpu.make_async_copy(xh.at[mcol.at[pl.ds(b * B, B)]], xb, sem)
            cpr = pltpu.make_async_copy(pnh.at[mrow.at[pl.ds(b * B, B)]], pnr, sem)
            cpc = pltpu.make_async_copy(pnh.at[mcol.at[pl.ds(b * B, B)]], pnc, sem)
            cpx.start()
            cpr.start()
            cpc.start()
            cpx.wait()
            cpr.wait()
            cpc.wait()
            for g in range(B // VEC):
                e0 = g * VEC
                ei = e0 + iota
                valid = (b * B + ei) < m
                prx = plsc.load_gather(pnr, [ei, jnp.full((VEC,), 0, jnp.int32)])
                pry = plsc.load_gather(pnr, [ei, jnp.full((VEC,), 1, jnp.int32)])
                prz = plsc.load_gather(pnr, [ei, jnp.full((VEC,), 2, jnp.int32)])
                nrx = plsc.load_gather(pnr, [ei, jnp.full((VEC,), 3, jnp.int32)])
                nry = plsc.load_gather(pnr, [ei, jnp.full((VEC,), 4, jnp.int32)])
                nrz = plsc.load_gather(pnr, [ei, jnp.full((VEC,), 5, jnp.int32)])
                pcx = plsc.load_gather(pnc, [ei, jnp.full((VEC,), 0, jnp.int32)])
                pcy = plsc.load_gather(pnc, [ei, jnp.full((VEC,), 1, jnp.int32)])
                pcz = plsc.load_gather(pnc, [ei, jnp.full((VEC,), 2, jnp.int32)])
                ncx = plsc.load_gather(pnc, [ei, jnp.full((VEC,), 3, jnp.int32)])
                ncy = plsc.load_gather(pnc, [ei, jnp.full((VEC,), 4, jnp.int32)])
                ncz = plsc.load_gather(pnc, [ei, jnp.full((VEC,), 5, jnp.int32)])
                dx = pcx - prx
                dy = pcy - pry
                dz = pcz - prz
                dist = dx * dx + dy * dy + dz * dz
                u1 = _surrogate(nrx, nry, nrz, dx, dy, dz)
                u2 = _surrogate(ncx, ncy, ncz, dx, dy, dz)
                u3 = _surrogate(nrx, nry, nrz, ncx, ncy, ncz)
                dsv = dsv + jnp.where(valid, dist, jnp.float32(0.0))
                fbuf[pl.ds(0, VEC)] = dist
                fbuf[pl.ds(VEC, VEC)] = u1
                fbuf[pl.ds(2 * VEC, VEC)] = u2
                fbuf[pl.ds(3 * VEC, VEC)] = u3
                for e in range(VEC):
                    eb = e0 + e

                    @pl.when(b * B + eb < m)
                    def _():
                        rsp = plsc.load_gather(
                            mrow, [jnp.full((VEC,), b * B + eb, jnp.int32)])
                        rl = rsp - lo
                        for j in range(D // VEC):
                            a = plsc.load_gather(accx, [rl, cj[j]])
                            xv = xb[eb, pl.ds(j * VEC, VEC)]
                            plsc.store_scatter(accx, [rl, cj[j]],
                                               jnp.maximum(a, xv))
                        basep = rl * VEC + iota
                        fv = plsc.load_gather(fbuf, [iota * VEC + e])
                        fa = plsc.load_gather(accp, [basep])
                        plsc.store_scatter(accp, [basep], jnp.maximum(fa, fv))
            return dsv

        nb = (m + (B - 1)) // B
        dsv = lax.fori_loop(jnp.int32(0), nb, batch_body, dsv)
        return kcv, dsv

    kcv, dsv = lax.fori_loop(
        jnp.int32(0), jnp.int32(EH // C), chunk_body,
        (jnp.zeros((VEC,), jnp.int32), jnp.zeros((VEC,), jnp.float32)))

    pltpu.sync_copy(accx, outx.at[pl.ds(c * NP + lo, RPW)])
    pltpu.sync_copy(accp.at[pl.ds(0, RPW * VEC)],
                    outp.at[pl.ds((c * NP + lo) * VEC, RPW * VEC)])
    svec[...] = dsv
    cvec[...] = kcv
    pltpu.sync_copy(svec, sums.at[pl.ds(w * VEC, VEC)])
    pltpu.sync_copy(cvec, counts.at[pl.ds(w * VEC, VEC)])


def _epi_body(x2_ref, p2_ref, s_ref, c_ref, o_ref):
    xm = jnp.maximum(x2_ref[0], x2_ref[1])
    pm = jnp.maximum(p2_ref[0], p2_ref[1])
    total = jnp.sum(s_ref[...], dtype=jnp.float32)
    ne = (jnp.sum(c_ref[...].astype(jnp.float32), dtype=jnp.float32)
          + jnp.float32(N))
    inv = ne / total
    dist = pm[:, 0:1] * inv
    u = pm[:, 1:4]
    cosv = -jnp.sign(u) * jnp.sqrt(jnp.abs(u))
    t = jnp.abs(cosv)
    p = jnp.sqrt(jnp.maximum(1.0 - t, 0.0)) * (
        1.5707288 + t * (-0.2121144 + t * (0.0742610 + t * (-0.0187293))))
    ang = jnp.where(cosv >= 0.0, p, jnp.float32(3.14159265358979) - p)
    o_ref[...] = jnp.concatenate([xm, dist, ang], axis=1)


@jax.jit
def _run(row32, col32, xpad, pn):
    mesh = plsc.VectorSubcoreMesh(core_axis_name="c", subcore_axis_name="s")
    outx, outp, sums, counts = pl.kernel(
        _sc_body,
        out_type=(
            jax.ShapeDtypeStruct((NC * NP, D), jnp.float32),
            jax.ShapeDtypeStruct((NC * NP * VEC,), jnp.float32),
            jax.ShapeDtypeStruct((NC * NS * VEC,), jnp.float32),
            jax.ShapeDtypeStruct((NC * NS * VEC,), jnp.int32),
        ),
        mesh=mesh,
        compiler_params=pltpu.CompilerParams(needs_layout_passes=False,
                                             use_tc_tiling_on_sc=False),
        scratch_types=[
            pltpu.VMEM((RPW, D), jnp.float32),
            pltpu.VMEM((RPW * VEC,), jnp.float32),
            pltpu.VMEM((C,), jnp.int32),
            pltpu.VMEM((C,), jnp.int32),
            pltpu.VMEM((C + VEC,), jnp.int32),
            pltpu.VMEM((C + VEC,), jnp.int32),
            pltpu.VMEM((B, D), jnp.float32),
            pltpu.VMEM((B, 8), jnp.float32),
            pltpu.VMEM((B, 8), jnp.float32),
            pltpu.VMEM((16 * VEC,), jnp.float32),
            pltpu.VMEM((VEC,), jnp.float32),
            pltpu.VMEM((VEC,), jnp.int32),
            pltpu.SemaphoreType.DMA,
        ],
    )(row32, col32, xpad, pn)
    out = pl.pallas_call(
        _epi_body,
        out_shape=jax.ShapeDtypeStruct((NP, 132), jnp.float32),
    )(outx.reshape(NC, NP, D), outp.reshape(NC, NP, VEC),
      sums.reshape(NC * NS, VEC), counts.reshape(NC * NS, VEC))
    return out[:N]


def kernel(x, pos, edge_index, norm, batch):
    row32 = edge_index[0].astype(jnp.int32)
    col32 = edge_index[1].astype(jnp.int32)
    x32 = x.astype(jnp.float32)
    pn = jnp.concatenate(
        [pos.astype(jnp.float32), norm.astype(jnp.float32),
         jnp.zeros((N, 2), jnp.float32)], axis=1)
    xpad = jnp.pad(x32, ((0, NP - N), (0, 0)))
    return _run(row32, col32, xpad, pn)
